# Initial kernel scaffold; baseline (speedup 1.0000x reference)
import jax
import jax.numpy as jnp
from jax.experimental import pallas as pl


def kernel(x, Wg, ng, Wd, nd, sWg, sng, sWd, snd, gate_norm_w, gate_w):
    raise NotImplementedError("write your pallas kernel here")


# speedup vs baseline: 7.7248x; 7.7248x over previous
import functools

import jax
import jax.numpy as jnp
from jax.experimental import pallas as pl
from jax.experimental.pallas import tpu as pltpu
from jax.experimental.pallas import tpu_sc as plsc

HD = 1024
ID = 512
NE = 64
CAP = 80
NSLOT = NE * CAP
TRASH = NSLOT
NT = 4096
TB = 256
EPS_GATE = 1e-6
EPS_BIT = 1e-8


def _rms(x, w, eps):
    return x * jax.lax.rsqrt(jnp.mean(x * x, axis=-1, keepdims=True) + eps) * w


def _ternary_bf16(w):
    sw = 1.0 / jnp.clip(jnp.mean(jnp.abs(w)), 1e-5, None)
    t = w * sw
    r = 1.0 / sw
    q = jnp.where(t > 0.5, r, jnp.where(t < -0.5, -r, 0.0))
    return q.astype(jnp.bfloat16)


def _bit_mlp(x, wg, ng, wd, nd):
    xn = _rms(x, ng, EPS_BIT)
    sx = 127.0 / jnp.clip(jnp.max(jnp.abs(xn), axis=-1, keepdims=True), 1e-5, None)
    xq = jnp.clip(jnp.round(xn * sx), -128.0, 127.0) / sx
    y = jax.lax.dot_general(
        xq.astype(jnp.bfloat16), _ternary_bf16(wg),
        (((1,), (1,)), ((), ())), preferred_element_type=jnp.float32)
    g = y[:, :ID]
    v = y[:, ID:]
    h = g * jax.nn.sigmoid(g) * v
    hn = _rms(h, nd, EPS_BIT)
    s2 = 127.0 / jnp.clip(jnp.max(jnp.abs(hn), axis=-1, keepdims=True), 1e-5, None)
    hq = jnp.clip(jnp.round(hn * s2), -128.0, 127.0) / s2
    return jax.lax.dot_general(
        hq.astype(jnp.bfloat16), _ternary_bf16(wd),
        (((1,), (1,)), ((), ())), preferred_element_type=jnp.float32)


def _router_body(x_ref, gnw_ref, gw_ref, dd_ref, dc_ref, w0_ref, w1_ref,
                 cnt_ref):
    step = pl.program_id(0)

    @pl.when(step == 0)
    def _():
        cnt_ref[...] = jnp.zeros_like(cnt_ref)

    x = x_ref[...]
    xn = x * jax.lax.rsqrt(jnp.mean(x * x, axis=-1, keepdims=True) + EPS_GATE)
    xn = xn * gnw_ref[...]
    logits = jax.lax.dot_general(
        xn.astype(jnp.bfloat16), gw_ref[...].astype(jnp.bfloat16),
        (((1,), (1,)), ((), ())),
        preferred_element_type=jnp.float32)
    m = jnp.max(logits, axis=-1, keepdims=True)
    p = jnp.exp(logits - m)
    p = p / jnp.sum(p, axis=-1, keepdims=True)

    idx = jax.lax.broadcasted_iota(jnp.int32, (TB, NE), 1)
    m1 = jnp.max(p, axis=-1, keepdims=True)
    i1 = jnp.min(jnp.where(p == m1, idx, NE), axis=-1, keepdims=True)
    o1 = idx == i1
    pm = jnp.where(o1, -1.0, p)
    m2 = jnp.max(pm, axis=-1, keepdims=True)
    i2 = jnp.min(jnp.where(pm == m2, idx, NE), axis=-1, keepdims=True)
    o2 = idx == i2

    a = (o1 | o2).astype(jnp.float32)
    c = a
    sh = 1
    while sh < TB:
        c = c + jnp.concatenate(
            [jnp.zeros((sh, NE), jnp.float32), c[:-sh]], axis=0)
        sh *= 2
    carry = cnt_ref[0:1, :]
    rank = carry + c - a
    r1 = jnp.sum(jnp.where(o1, rank, 0.0), axis=-1, keepdims=True)
    r2 = jnp.sum(jnp.where(o2, rank, 0.0), axis=-1, keepdims=True)
    in1 = r1 < CAP
    in2 = r2 < CAP
    d1 = i1 * CAP + r1.astype(jnp.int32)
    d2 = i2 * CAP + r2.astype(jnp.int32)
    dd_ref[...] = jnp.concatenate(
        [jnp.where(in1, d1, TRASH), jnp.where(in2, d2, TRASH)], axis=1)
    dc_ref[...] = jnp.concatenate(
        [jnp.where(in1, d1, 0), jnp.where(in2, d2, 0)], axis=1)
    w0_ref[...] = jnp.broadcast_to(jnp.where(in1, m1, 0.0), (TB, 16))
    w1_ref[...] = jnp.broadcast_to(jnp.where(in2, m2, 0.0), (TB, 16))
    cnt_ref[0:1, :] = carry + jnp.sum(a, axis=0, keepdims=True)


def _router(xf, gnw, gw):
    return pl.pallas_call(
        _router_body,
        grid=(NT // TB,),
        in_specs=[
            pl.BlockSpec((TB, HD), lambda i: (i, 0)),
            pl.BlockSpec((1, HD), lambda i: (0, 0)),
            pl.BlockSpec((NE, HD), lambda i: (0, 0)),
        ],
        out_specs=[
            pl.BlockSpec((TB, 2), lambda i: (i, 0)),
            pl.BlockSpec((TB, 2), lambda i: (i, 0)),
            pl.BlockSpec((TB, 16), lambda i: (i, 0)),
            pl.BlockSpec((TB, 16), lambda i: (i, 0)),
        ],
        out_shape=[
            jax.ShapeDtypeStruct((NT, 2), jnp.int32),
            jax.ShapeDtypeStruct((NT, 2), jnp.int32),
            jax.ShapeDtypeStruct((NT, 16), jnp.float32),
            jax.ShapeDtypeStruct((NT, 16), jnp.float32),
        ],
        scratch_shapes=[pltpu.VMEM((8, NE), jnp.float32)],
    )(xf, gnw, gw)


def _expert_body(xg_ref, wg_ref, ng_ref, wd_ref, nd_ref, yg_ref):
    yg_ref[...] = _bit_mlp(
        xg_ref[...], wg_ref[0], ng_ref[0], wd_ref[0], nd_ref[0])


def _experts(xg, Wg, ng, Wd, nd):
    return pl.pallas_call(
        _expert_body,
        grid=(NE,),
        in_specs=[
            pl.BlockSpec((CAP, HD), lambda e: (e, 0)),
            pl.BlockSpec((1, 2 * ID, HD), lambda e: (e, 0, 0)),
            pl.BlockSpec((1, 1, HD), lambda e: (e, 0, 0)),
            pl.BlockSpec((1, HD, ID), lambda e: (e, 0, 0)),
            pl.BlockSpec((1, 1, ID), lambda e: (e, 0, 0)),
        ],
        out_specs=pl.BlockSpec((CAP, HD), lambda e: (e, 0)),
        out_shape=jax.ShapeDtypeStruct((NSLOT, HD), jnp.float32),
    )(xg, Wg, ng.reshape(NE, 1, HD), Wd, nd.reshape(NE, 1, ID))


def _shared_body(x_ref, wg_ref, ng_ref, wd_ref, nd_ref, y_ref):
    y_ref[...] = _bit_mlp(
        x_ref[...], wg_ref[...], ng_ref[...], wd_ref[...], nd_ref[...])


def _shared(xf, sWg, sng, sWd, snd):
    blk = 512
    return pl.pallas_call(
        _shared_body,
        grid=(NT // blk,),
        in_specs=[
            pl.BlockSpec((blk, HD), lambda i: (i, 0)),
            pl.BlockSpec((2 * ID, HD), lambda i: (0, 0)),
            pl.BlockSpec((1, HD), lambda i: (0, 0)),
            pl.BlockSpec((HD, ID), lambda i: (0, 0)),
            pl.BlockSpec((1, ID), lambda i: (0, 0)),
        ],
        out_specs=pl.BlockSpec((blk, HD), lambda i: (i, 0)),
        out_shape=jax.ShapeDtypeStruct((NT, HD), jnp.float32),
    )(xf, sWg, sng, sWd, snd)


_NW = 32
_TPW = NT // _NW
_CH = 32

_SC_MESH = dict(core_axis_name="c", subcore_axis_name="s",
                num_cores=2, num_subcores=16)


def _worker_id():
    return jax.lax.axis_index("s") * 2 + jax.lax.axis_index("c")


def _dispatch_body(x_hbm, d0_hbm, d1_hbm, xg_hbm, rows_v, i0_v, i1_v, sem):
    wid = _worker_id()

    def chunk(i, carry):
        base = pl.multiple_of(wid * _TPW + i * _CH, _CH)
        pltpu.sync_copy(x_hbm.at[pl.ds(base, _CH)], rows_v)
        pltpu.sync_copy(d0_hbm.at[pl.ds(base, _CH)], i0_v)
        pltpu.sync_copy(d1_hbm.at[pl.ds(base, _CH)], i1_v)
        pltpu.async_copy(rows_v, xg_hbm.at[i0_v], sem).wait()
        pltpu.async_copy(rows_v, xg_hbm.at[i1_v], sem).wait()
        return carry

    jax.lax.fori_loop(0, _TPW // _CH, chunk, 0)


def _dispatch(xf, dd0, dd1):
    f = functools.partial(
        pl.kernel,
        out_type=jax.ShapeDtypeStruct((NSLOT + 8, HD), jnp.float32),
        mesh=plsc.VectorSubcoreMesh(**_SC_MESH),
        scratch_types=[
            pltpu.VMEM((_CH, HD), jnp.float32),
            pltpu.VMEM((_CH,), jnp.int32),
            pltpu.VMEM((_CH,), jnp.int32),
            pltpu.SemaphoreType.DMA,
        ],
    )(_dispatch_body)
    return f(xf, dd0, dd1)


def _combine_body(sh_hbm, yg_hbm, d0_hbm, d1_hbm, w0_hbm, w1_hbm, out_hbm,
                  s_v, y0_v, y1_v, i0_v, i1_v, w0_v, w1_v, sem):
    wid = _worker_id()

    def chunk(i, carry):
        base = pl.multiple_of(wid * _TPW + i * _CH, _CH)
        pltpu.sync_copy(d0_hbm.at[pl.ds(base, _CH)], i0_v)
        pltpu.sync_copy(d1_hbm.at[pl.ds(base, _CH)], i1_v)
        pltpu.sync_copy(w0_hbm.at[pl.ds(base, _CH)], w0_v)
        pltpu.sync_copy(w1_hbm.at[pl.ds(base, _CH)], w1_v)
        pltpu.sync_copy(sh_hbm.at[pl.ds(base, _CH)], s_v)
        pltpu.async_copy(yg_hbm.at[i0_v], y0_v, sem).wait()
        pltpu.async_copy(yg_hbm.at[i1_v], y1_v, sem).wait()

        def token(t, c2):
            w0s = w0_v[t, :]
            w1s = w1_v[t, :]
            zero = jnp.zeros((16,), jnp.float32)

            def grp4(g4, c3):
                for u in range(4):
                    off = (g4 * 4 + u) * 16
                    acc = s_v[t, pl.ds(off, 16)]
                    y0 = y0_v[t, pl.ds(off, 16)]
                    y1 = y1_v[t, pl.ds(off, 16)]
                    acc = acc + jnp.where(w0s != 0.0, w0s * y0, zero)
                    acc = acc + jnp.where(w1s != 0.0, w1s * y1, zero)
                    s_v[t, pl.ds(off, 16)] = acc
                return c3

            jax.lax.fori_loop(0, HD // 64, grp4, 0)
            return c2

        jax.lax.fori_loop(0, _CH, token, 0)
        pltpu.sync_copy(s_v, out_hbm.at[pl.ds(base, _CH)])
        return carry

    jax.lax.fori_loop(0, _TPW // _CH, chunk, 0)


def _combine(sh, yg, dc0, dc1, w0, w1):
    f = functools.partial(
        pl.kernel,
        out_type=jax.ShapeDtypeStruct((NT, HD), jnp.float32),
        mesh=plsc.VectorSubcoreMesh(**_SC_MESH),
        scratch_types=[
            pltpu.VMEM((_CH, HD), jnp.float32),
            pltpu.VMEM((_CH, HD), jnp.float32),
            pltpu.VMEM((_CH, HD), jnp.float32),
            pltpu.VMEM((_CH,), jnp.int32),
            pltpu.VMEM((_CH,), jnp.int32),
            pltpu.VMEM((_CH, 16), jnp.float32),
            pltpu.VMEM((_CH, 16), jnp.float32),
            pltpu.SemaphoreType.DMA,
        ],
    )(_combine_body)
    return f(sh, yg, dc0, dc1, w0, w1)


def kernel(x, Wg, ng, Wd, nd, sWg, sng, sWd, snd, gate_norm_w, gate_w):
    xf = x.reshape(-1, HD)
    dd, dc, w0b, w1b = _router(xf, gate_norm_w.reshape(1, HD), gate_w)
    xg = _dispatch(xf, dd[:, 0], dd[:, 1])
    yg = _experts(xg, Wg, ng, Wd, nd)
    sh = _shared(xf, sWg, sng.reshape(1, HD), sWd, snd.reshape(1, ID))
    out = _combine(sh, yg, dc[:, 0], dc[:, 1], w0b, w1b)
    return out.reshape(x.shape[0], x.shape[1], HD)


# speedup vs baseline: 8.4528x; 1.0942x over previous
import functools

import jax
import jax.numpy as jnp
from jax.experimental import pallas as pl
from jax.experimental.pallas import tpu as pltpu
from jax.experimental.pallas import tpu_sc as plsc

HD = 1024
ID = 512
NE = 64
CAP = 80
NSLOT = NE * CAP
TRASH = NSLOT
NT = 4096
TB = 256
EPS_GATE = 1e-6
EPS_BIT = 1e-8


def _rms(x, w, eps):
    return x * jax.lax.rsqrt(jnp.mean(x * x, axis=-1, keepdims=True) + eps) * w


def _ternary_bf16(w):
    m = jnp.sum(jnp.sum(jnp.abs(w), axis=0, keepdims=True), axis=1,
                keepdims=True) * (1.0 / (w.shape[0] * w.shape[1]))
    sw = 1.0 / jnp.clip(m, 1e-5, None)
    t = w * sw
    r = 1.0 / sw
    q = jnp.where(t > 0.5, r, jnp.where(t < -0.5, -r, 0.0))
    return q.astype(jnp.bfloat16)


def _bit_mlp(x, wg, ng, wd, nd):
    xn = _rms(x, ng, EPS_BIT)
    sx = 127.0 / jnp.clip(jnp.max(jnp.abs(xn), axis=-1, keepdims=True), 1e-5, None)
    xq = jnp.clip(jnp.round(xn * sx), -128.0, 127.0) / sx
    y = jax.lax.dot_general(
        xq.astype(jnp.bfloat16), _ternary_bf16(wg),
        (((1,), (1,)), ((), ())), preferred_element_type=jnp.float32)
    g = y[:, :ID]
    v = y[:, ID:]
    h = g * jax.nn.sigmoid(g) * v
    hn = _rms(h, nd, EPS_BIT)
    s2 = 127.0 / jnp.clip(jnp.max(jnp.abs(hn), axis=-1, keepdims=True), 1e-5, None)
    hq = jnp.clip(jnp.round(hn * s2), -128.0, 127.0) / s2
    return jax.lax.dot_general(
        hq.astype(jnp.bfloat16), _ternary_bf16(wd),
        (((1,), (1,)), ((), ())), preferred_element_type=jnp.float32)


def _router_body(x_ref, gnw_ref, gw_ref, dd_ref, dc_ref, w0_ref, w1_ref,
                 cnt_ref):
    step = pl.program_id(0)

    @pl.when(step == 0)
    def _():
        cnt_ref[...] = jnp.zeros_like(cnt_ref)

    x = x_ref[...]
    xn = x * jax.lax.rsqrt(jnp.mean(x * x, axis=-1, keepdims=True) + EPS_GATE)
    xn = xn * gnw_ref[...]
    logits = jax.lax.dot_general(
        xn.astype(jnp.bfloat16), gw_ref[...].astype(jnp.bfloat16),
        (((1,), (1,)), ((), ())),
        preferred_element_type=jnp.float32)
    m = jnp.max(logits, axis=-1, keepdims=True)
    p = jnp.exp(logits - m)
    p = p / jnp.sum(p, axis=-1, keepdims=True)

    idx = jax.lax.broadcasted_iota(jnp.int32, (TB, NE), 1)
    m1 = jnp.max(p, axis=-1, keepdims=True)
    i1 = jnp.min(jnp.where(p == m1, idx, NE), axis=-1, keepdims=True)
    o1 = idx == i1
    pm = jnp.where(o1, -1.0, p)
    m2 = jnp.max(pm, axis=-1, keepdims=True)
    i2 = jnp.min(jnp.where(pm == m2, idx, NE), axis=-1, keepdims=True)
    o2 = idx == i2

    a = (o1 | o2).astype(jnp.float32)
    c = a
    sh = 1
    while sh < TB:
        c = c + jnp.concatenate(
            [jnp.zeros((sh, NE), jnp.float32), c[:-sh]], axis=0)
        sh *= 2
    carry = cnt_ref[0:1, :]
    rank = carry + c - a
    r1 = jnp.sum(jnp.where(o1, rank, 0.0), axis=-1, keepdims=True)
    r2 = jnp.sum(jnp.where(o2, rank, 0.0), axis=-1, keepdims=True)
    in1 = r1 < CAP
    in2 = r2 < CAP
    d1 = i1 * CAP + r1.astype(jnp.int32)
    d2 = i2 * CAP + r2.astype(jnp.int32)
    dd_ref[...] = jnp.concatenate(
        [jnp.where(in1, d1, TRASH), jnp.where(in2, d2, TRASH)], axis=1)
    dc_ref[...] = jnp.concatenate(
        [jnp.where(in1, d1, 0), jnp.where(in2, d2, 0)], axis=1)
    w0_ref[...] = jnp.broadcast_to(jnp.where(in1, m1, 0.0), (TB, 16))
    w1_ref[...] = jnp.broadcast_to(jnp.where(in2, m2, 0.0), (TB, 16))
    cnt_ref[0:1, :] = carry + jnp.sum(a, axis=0, keepdims=True)


def _router(xf, gnw, gw):
    return pl.pallas_call(
        _router_body,
        grid=(NT // TB,),
        in_specs=[
            pl.BlockSpec((TB, HD), lambda i: (i, 0)),
            pl.BlockSpec((1, HD), lambda i: (0, 0)),
            pl.BlockSpec((NE, HD), lambda i: (0, 0)),
        ],
        out_specs=[
            pl.BlockSpec((TB, 2), lambda i: (i, 0)),
            pl.BlockSpec((TB, 2), lambda i: (i, 0)),
            pl.BlockSpec((TB, 16), lambda i: (i, 0)),
            pl.BlockSpec((TB, 16), lambda i: (i, 0)),
        ],
        out_shape=[
            jax.ShapeDtypeStruct((NT, 2), jnp.int32),
            jax.ShapeDtypeStruct((NT, 2), jnp.int32),
            jax.ShapeDtypeStruct((NT, 16), jnp.float32),
            jax.ShapeDtypeStruct((NT, 16), jnp.float32),
        ],
        scratch_shapes=[pltpu.VMEM((8, NE), jnp.float32)],
    )(xf, gnw, gw)


def _expert_body(xg_ref, wg_ref, ng_ref, wd_ref, nd_ref, yg_ref):
    yg_ref[...] = _bit_mlp(
        xg_ref[...], wg_ref[0], ng_ref[0], wd_ref[0], nd_ref[0])


def _experts(xg, Wg, ng, Wd, nd):
    return pl.pallas_call(
        _expert_body,
        grid=(NE,),
        in_specs=[
            pl.BlockSpec((CAP, HD), lambda e: (e, 0)),
            pl.BlockSpec((1, 2 * ID, HD), lambda e: (e, 0, 0)),
            pl.BlockSpec((1, 1, HD), lambda e: (e, 0, 0)),
            pl.BlockSpec((1, HD, ID), lambda e: (e, 0, 0)),
            pl.BlockSpec((1, 1, ID), lambda e: (e, 0, 0)),
        ],
        out_specs=pl.BlockSpec((CAP, HD), lambda e: (e, 0)),
        out_shape=jax.ShapeDtypeStruct((NSLOT, HD), jnp.float32),
    )(xg, Wg, ng.reshape(NE, 1, HD), Wd, nd.reshape(NE, 1, ID))


def _quant_body(wg_ref, wd_ref, wgq_ref, wdq_ref):
    wgq_ref[...] = _ternary_bf16(wg_ref[...])
    wdq_ref[...] = _ternary_bf16(wd_ref[...])


def _quant_shared(sWg, sWd):
    return pl.pallas_call(
        _quant_body,
        out_shape=[
            jax.ShapeDtypeStruct((2 * ID, HD), jnp.bfloat16),
            jax.ShapeDtypeStruct((HD, ID), jnp.bfloat16),
        ],
    )(sWg, sWd)


def _shared_body(x_ref, wgq_ref, ng_ref, wdq_ref, nd_ref, y_ref):
    x = x_ref[...]
    ng = ng_ref[...]
    nd = nd_ref[...]
    xn = _rms(x, ng, EPS_BIT)
    sx = 127.0 / jnp.clip(jnp.max(jnp.abs(xn), axis=-1, keepdims=True), 1e-5, None)
    xq = jnp.clip(jnp.round(xn * sx), -128.0, 127.0) / sx
    y = jax.lax.dot_general(
        xq.astype(jnp.bfloat16), wgq_ref[...],
        (((1,), (1,)), ((), ())), preferred_element_type=jnp.float32)
    g = y[:, :ID]
    v = y[:, ID:]
    h = g * jax.nn.sigmoid(g) * v
    hn = _rms(h, nd, EPS_BIT)
    s2 = 127.0 / jnp.clip(jnp.max(jnp.abs(hn), axis=-1, keepdims=True), 1e-5, None)
    hq = jnp.clip(jnp.round(hn * s2), -128.0, 127.0) / s2
    y_ref[...] = jax.lax.dot_general(
        hq.astype(jnp.bfloat16), wdq_ref[...],
        (((1,), (1,)), ((), ())), preferred_element_type=jnp.float32)


def _shared(xf, sWg, sng, sWd, snd):
    blk = 512
    wgq, wdq = _quant_shared(sWg, sWd)
    return pl.pallas_call(
        _shared_body,
        grid=(NT // blk,),
        in_specs=[
            pl.BlockSpec((blk, HD), lambda i: (i, 0)),
            pl.BlockSpec((2 * ID, HD), lambda i: (0, 0)),
            pl.BlockSpec((1, HD), lambda i: (0, 0)),
            pl.BlockSpec((HD, ID), lambda i: (0, 0)),
            pl.BlockSpec((1, ID), lambda i: (0, 0)),
        ],
        out_specs=pl.BlockSpec((blk, HD), lambda i: (i, 0)),
        out_shape=jax.ShapeDtypeStruct((NT, HD), jnp.float32),
    )(xf, wgq, sng, wdq, snd)


_NW = 32
_TPW = NT // _NW
_CH = 32

_SC_MESH = dict(core_axis_name="c", subcore_axis_name="s",
                num_cores=2, num_subcores=16)


def _worker_id():
    return jax.lax.axis_index("s") * 2 + jax.lax.axis_index("c")


def _dispatch_body(x_hbm, d0_hbm, d1_hbm, xg_hbm, rows_v, i0_v, i1_v, sem):
    wid = _worker_id()

    def chunk(i, carry):
        base = pl.multiple_of(wid * _TPW + i * _CH, _CH)
        pltpu.sync_copy(x_hbm.at[pl.ds(base, _CH)], rows_v)
        pltpu.sync_copy(d0_hbm.at[pl.ds(base, _CH)], i0_v)
        pltpu.sync_copy(d1_hbm.at[pl.ds(base, _CH)], i1_v)
        pltpu.async_copy(rows_v, xg_hbm.at[i0_v], sem).wait()
        pltpu.async_copy(rows_v, xg_hbm.at[i1_v], sem).wait()
        return carry

    jax.lax.fori_loop(0, _TPW // _CH, chunk, 0)


def _dispatch(xf, dd0, dd1):
    f = functools.partial(
        pl.kernel,
        out_type=jax.ShapeDtypeStruct((NSLOT + 8, HD), jnp.float32),
        mesh=plsc.VectorSubcoreMesh(**_SC_MESH),
        scratch_types=[
            pltpu.VMEM((_CH, HD), jnp.float32),
            pltpu.VMEM((_CH,), jnp.int32),
            pltpu.VMEM((_CH,), jnp.int32),
            pltpu.SemaphoreType.DMA,
        ],
    )(_dispatch_body)
    return f(xf, dd0, dd1)


def _combine_body(sh_hbm, yg_hbm, d0_hbm, d1_hbm, w0_hbm, w1_hbm, out_hbm,
                  s_v, y0_v, y1_v, i0_v, i1_v, w0_v, w1_v, sem):
    wid = _worker_id()

    def chunk(i, carry):
        base = pl.multiple_of(wid * _TPW + i * _CH, _CH)
        pltpu.sync_copy(d0_hbm.at[pl.ds(base, _CH)], i0_v)
        pltpu.sync_copy(d1_hbm.at[pl.ds(base, _CH)], i1_v)
        pltpu.sync_copy(w0_hbm.at[pl.ds(base, _CH)], w0_v)
        pltpu.sync_copy(w1_hbm.at[pl.ds(base, _CH)], w1_v)
        pltpu.sync_copy(sh_hbm.at[pl.ds(base, _CH)], s_v)
        pltpu.async_copy(yg_hbm.at[i0_v], y0_v, sem).wait()
        pltpu.async_copy(yg_hbm.at[i1_v], y1_v, sem).wait()

        def token(t, c2):
            w0s = w0_v[t, :]
            w1s = w1_v[t, :]
            zero = jnp.zeros((16,), jnp.float32)

            def grp4(g4, c3):
                for u in range(4):
                    off = (g4 * 4 + u) * 16
                    acc = s_v[t, pl.ds(off, 16)]
                    y0 = y0_v[t, pl.ds(off, 16)]
                    y1 = y1_v[t, pl.ds(off, 16)]
                    acc = acc + jnp.where(w0s != 0.0, w0s * y0, zero)
                    acc = acc + jnp.where(w1s != 0.0, w1s * y1, zero)
                    s_v[t, pl.ds(off, 16)] = acc
                return c3

            jax.lax.fori_loop(0, HD // 64, grp4, 0)
            return c2

        jax.lax.fori_loop(0, _CH, token, 0)
        pltpu.sync_copy(s_v, out_hbm.at[pl.ds(base, _CH)])
        return carry

    jax.lax.fori_loop(0, _TPW // _CH, chunk, 0)


def _combine(sh, yg, dc0, dc1, w0, w1):
    f = functools.partial(
        pl.kernel,
        out_type=jax.ShapeDtypeStruct((NT, HD), jnp.float32),
        mesh=plsc.VectorSubcoreMesh(**_SC_MESH),
        scratch_types=[
            pltpu.VMEM((_CH, HD), jnp.float32),
            pltpu.VMEM((_CH, HD), jnp.float32),
            pltpu.VMEM((_CH, HD), jnp.float32),
            pltpu.VMEM((_CH,), jnp.int32),
            pltpu.VMEM((_CH,), jnp.int32),
            pltpu.VMEM((_CH, 16), jnp.float32),
            pltpu.VMEM((_CH, 16), jnp.float32),
            pltpu.SemaphoreType.DMA,
        ],
    )(_combine_body)
    return f(sh, yg, dc0, dc1, w0, w1)


def kernel(x, Wg, ng, Wd, nd, sWg, sng, sWd, snd, gate_norm_w, gate_w):
    xf = x.reshape(-1, HD)
    dd, dc, w0b, w1b = _router(xf, gate_norm_w.reshape(1, HD), gate_w)
    xg = _dispatch(xf, dd[:, 0], dd[:, 1])
    yg = _experts(xg, Wg, ng, Wd, nd)
    sh = _shared(xf, sWg, sng.reshape(1, HD), sWd, snd.reshape(1, ID))
    out = _combine(sh, yg, dc[:, 0], dc[:, 1], w0b, w1b)
    return out.reshape(x.shape[0], x.shape[1], HD)


# speedup vs baseline: 15.3661x; 1.8179x over previous
import functools

import jax
import jax.numpy as jnp
from jax.experimental import pallas as pl
from jax.experimental.pallas import tpu as pltpu
from jax.experimental.pallas import tpu_sc as plsc

HD = 1024
ID = 512
NE = 64
CAP = 80
NSLOT = NE * CAP
TRASH = NSLOT
NT = 4096
TB = 256
EPS_GATE = 1e-6
EPS_BIT = 1e-8


def _rms(x, w, eps):
    return x * jax.lax.rsqrt(jnp.mean(x * x, axis=-1, keepdims=True) + eps) * w


def _ternary_bf16(w):
    m = jnp.sum(jnp.sum(jnp.abs(w), axis=0, keepdims=True), axis=1,
                keepdims=True) * (1.0 / (w.shape[0] * w.shape[1]))
    sw = 1.0 / jnp.clip(m, 1e-5, None)
    t = w * sw
    r = 1.0 / sw
    q = jnp.where(t > 0.5, r, jnp.where(t < -0.5, -r, 0.0))
    return q.astype(jnp.bfloat16)


def _bit_mlp(x, wg, ng, wd, nd):
    xn = _rms(x, ng, EPS_BIT)
    sx = 127.0 / jnp.clip(jnp.max(jnp.abs(xn), axis=-1, keepdims=True), 1e-5, None)
    xq = jnp.clip(jnp.round(xn * sx), -128.0, 127.0) / sx
    y = jax.lax.dot_general(
        xq.astype(jnp.bfloat16), _ternary_bf16(wg),
        (((1,), (1,)), ((), ())), preferred_element_type=jnp.float32)
    g = y[:, :ID]
    v = y[:, ID:]
    h = g * jax.nn.sigmoid(g) * v
    hn = _rms(h, nd, EPS_BIT)
    s2 = 127.0 / jnp.clip(jnp.max(jnp.abs(hn), axis=-1, keepdims=True), 1e-5, None)
    hq = jnp.clip(jnp.round(hn * s2), -128.0, 127.0) / s2
    return jax.lax.dot_general(
        hq.astype(jnp.bfloat16), _ternary_bf16(wd),
        (((1,), (1,)), ((), ())), preferred_element_type=jnp.float32)


def _router_body(x_ref, gnw_ref, gw_ref, dd_ref, dc_ref, wt_ref, cnt_ref):
    step = pl.program_id(0)

    @pl.when(step == 0)
    def _():
        cnt_ref[...] = jnp.zeros_like(cnt_ref)

    x = x_ref[...]
    xn = x * jax.lax.rsqrt(jnp.mean(x * x, axis=-1, keepdims=True) + EPS_GATE)
    xn = xn * gnw_ref[...]
    logits = jax.lax.dot_general(
        xn.astype(jnp.bfloat16), gw_ref[...].astype(jnp.bfloat16),
        (((1,), (1,)), ((), ())),
        preferred_element_type=jnp.float32)
    m = jnp.max(logits, axis=-1, keepdims=True)
    p = jnp.exp(logits - m)
    p = p / jnp.sum(p, axis=-1, keepdims=True)

    idx = jax.lax.broadcasted_iota(jnp.int32, (TB, NE), 1)
    m1 = jnp.max(p, axis=-1, keepdims=True)
    i1 = jnp.min(jnp.where(p == m1, idx, NE), axis=-1, keepdims=True)
    o1 = idx == i1
    pm = jnp.where(o1, -1.0, p)
    m2 = jnp.max(pm, axis=-1, keepdims=True)
    i2 = jnp.min(jnp.where(pm == m2, idx, NE), axis=-1, keepdims=True)
    o2 = idx == i2

    a = (o1 | o2).astype(jnp.float32)
    c = a
    sh = 1
    while sh < TB:
        c = c + jnp.concatenate(
            [jnp.zeros((sh, NE), jnp.float32), c[:-sh]], axis=0)
        sh *= 2
    carry = cnt_ref[0:1, :]
    rank = carry + c - a
    r1 = jnp.sum(jnp.where(o1, rank, 0.0), axis=-1, keepdims=True)
    r2 = jnp.sum(jnp.where(o2, rank, 0.0), axis=-1, keepdims=True)
    in1 = r1 < CAP
    in2 = r2 < CAP
    d1 = i1 * CAP + r1.astype(jnp.int32)
    d2 = i2 * CAP + r2.astype(jnp.int32)
    tok = (jax.lax.broadcasted_iota(jnp.int32, (TB, 1), 0) + step * TB)
    dd_ref[...] = jnp.concatenate(
        [jnp.where(in1, d1, TRASH + tok), jnp.where(in2, d2, TRASH + tok)],
        axis=1)
    dc_ref[...] = jnp.concatenate(
        [jnp.where(in1, d1, tok), jnp.where(in2, d2, tok)], axis=1)
    wt_ref[...] = jnp.concatenate(
        [jnp.where(in1, m1, 0.0), jnp.where(in2, m2, 0.0)], axis=1)
    cnt_ref[0:1, :] = carry + jnp.sum(a, axis=0, keepdims=True)


def _router(xf, gnw, gw):
    return pl.pallas_call(
        _router_body,
        grid=(NT // TB,),
        in_specs=[
            pl.BlockSpec((TB, HD), lambda i: (i, 0)),
            pl.BlockSpec((1, HD), lambda i: (0, 0)),
            pl.BlockSpec((NE, HD), lambda i: (0, 0)),
        ],
        out_specs=[
            pl.BlockSpec((TB, 2), lambda i: (i, 0)),
            pl.BlockSpec((TB, 2), lambda i: (i, 0)),
            pl.BlockSpec((TB, 2), lambda i: (i, 0)),
        ],
        out_shape=[
            jax.ShapeDtypeStruct((NT, 2), jnp.int32),
            jax.ShapeDtypeStruct((NT, 2), jnp.int32),
            jax.ShapeDtypeStruct((NT, 2), jnp.float32),
        ],
        scratch_shapes=[pltpu.VMEM((8, NE), jnp.float32)],
    )(xf, gnw, gw)


def _expert_body(xg_ref, wg_ref, ng_ref, wd_ref, nd_ref, yg_ref):
    yg_ref[...] = _bit_mlp(
        xg_ref[...], wg_ref[0], ng_ref[0], wd_ref[0], nd_ref[0])


def _experts(xg, Wg, ng, Wd, nd):
    return pl.pallas_call(
        _expert_body,
        grid=(NE,),
        in_specs=[
            pl.BlockSpec((CAP, HD), lambda e: (e, 0)),
            pl.BlockSpec((1, 2 * ID, HD), lambda e: (e, 0, 0)),
            pl.BlockSpec((1, 1, HD), lambda e: (e, 0, 0)),
            pl.BlockSpec((1, HD, ID), lambda e: (e, 0, 0)),
            pl.BlockSpec((1, 1, ID), lambda e: (e, 0, 0)),
        ],
        out_specs=pl.BlockSpec((CAP, HD), lambda e: (e, 0)),
        out_shape=jax.ShapeDtypeStruct((NSLOT, HD), jnp.float32),
    )(xg, Wg, ng.reshape(NE, 1, HD), Wd, nd.reshape(NE, 1, ID))


def _quant_body(wg_ref, wd_ref, wgq_ref, wdq_ref):
    wgq_ref[...] = _ternary_bf16(wg_ref[...])
    wdq_ref[...] = _ternary_bf16(wd_ref[...])


def _quant_shared(sWg, sWd):
    return pl.pallas_call(
        _quant_body,
        out_shape=[
            jax.ShapeDtypeStruct((2 * ID, HD), jnp.bfloat16),
            jax.ShapeDtypeStruct((HD, ID), jnp.bfloat16),
        ],
    )(sWg, sWd)


def _shared_body(x_ref, wgq_ref, ng_ref, wdq_ref, nd_ref,
                 y0_ref, y1_ref, wt_ref, y_ref):
    x = x_ref[...]
    ng = ng_ref[...]
    nd = nd_ref[...]
    xn = _rms(x, ng, EPS_BIT)
    sx = 127.0 / jnp.clip(jnp.max(jnp.abs(xn), axis=-1, keepdims=True), 1e-5, None)
    xq = jnp.clip(jnp.round(xn * sx), -128.0, 127.0) / sx
    y = jax.lax.dot_general(
        xq.astype(jnp.bfloat16), wgq_ref[...],
        (((1,), (1,)), ((), ())), preferred_element_type=jnp.float32)
    g = y[:, :ID]
    v = y[:, ID:]
    h = g * jax.nn.sigmoid(g) * v
    hn = _rms(h, nd, EPS_BIT)
    s2 = 127.0 / jnp.clip(jnp.max(jnp.abs(hn), axis=-1, keepdims=True), 1e-5, None)
    hq = jnp.clip(jnp.round(hn * s2), -128.0, 127.0) / s2
    sh = jax.lax.dot_general(
        hq.astype(jnp.bfloat16), wdq_ref[...],
        (((1,), (1,)), ((), ())), preferred_element_type=jnp.float32)
    w0 = wt_ref[:, 0:1]
    w1 = wt_ref[:, 1:2]
    z = jnp.zeros_like(sh)
    o = sh + jnp.where(w0 != 0.0, w0 * y0_ref[...], z)
    y_ref[...] = o + jnp.where(w1 != 0.0, w1 * y1_ref[...], z)


def _shared_combine(xf, sWg, sng, sWd, snd, y0, y1, wt):
    blk = 512
    wgq, wdq = _quant_shared(sWg, sWd)
    return pl.pallas_call(
        _shared_body,
        grid=(NT // blk,),
        in_specs=[
            pl.BlockSpec((blk, HD), lambda i: (i, 0)),
            pl.BlockSpec((2 * ID, HD), lambda i: (0, 0)),
            pl.BlockSpec((1, HD), lambda i: (0, 0)),
            pl.BlockSpec((HD, ID), lambda i: (0, 0)),
            pl.BlockSpec((1, ID), lambda i: (0, 0)),
            pl.BlockSpec((blk, HD), lambda i: (i, 0)),
            pl.BlockSpec((blk, HD), lambda i: (i, 0)),
            pl.BlockSpec((blk, 2), lambda i: (i, 0)),
        ],
        out_specs=pl.BlockSpec((blk, HD), lambda i: (i, 0)),
        out_shape=jax.ShapeDtypeStruct((NT, HD), jnp.float32),
    )(xf, wgq, sng, wdq, snd, y0, y1, wt)


_NW = 32
_TPW = NT // _NW
_CH = 32

_SC_MESH = dict(core_axis_name="c", subcore_axis_name="s",
                num_cores=2, num_subcores=16)


def _worker_id():
    return jax.lax.axis_index("s") * 2 + jax.lax.axis_index("c")


def _dispatch_body(x_hbm, d0_hbm, d1_hbm, xg_hbm, rows_v, i0_v, i1_v, sem):
    wid = _worker_id()

    def chunk(i, carry):
        base = pl.multiple_of(wid * _TPW + i * _CH, _CH)
        pltpu.sync_copy(x_hbm.at[pl.ds(base, _CH)], rows_v)
        pltpu.sync_copy(d0_hbm.at[pl.ds(base, _CH)], i0_v)
        pltpu.sync_copy(d1_hbm.at[pl.ds(base, _CH)], i1_v)
        c0 = pltpu.async_copy(rows_v, xg_hbm.at[i0_v], sem)
        c1 = pltpu.async_copy(rows_v, xg_hbm.at[i1_v], sem)
        c0.wait()
        c1.wait()
        return carry

    jax.lax.fori_loop(0, _TPW // _CH, chunk, 0)


def _dispatch(xf, dd0, dd1):
    f = functools.partial(
        pl.kernel,
        out_type=jax.ShapeDtypeStruct((NSLOT + NT, HD), jnp.float32),
        mesh=plsc.VectorSubcoreMesh(**_SC_MESH),
        scratch_types=[
            pltpu.VMEM((_CH, HD), jnp.float32),
            pltpu.VMEM((_CH,), jnp.int32),
            pltpu.VMEM((_CH,), jnp.int32),
            pltpu.SemaphoreType.DMA,
        ],
    )(_dispatch_body)
    return f(xf, dd0, dd1)


def _gather_body(yg_hbm, d0_hbm, d1_hbm, y0_hbm, y1_hbm,
                 b0_v, b1_v, i0_v, i1_v, sem):
    wid = _worker_id()

    def chunk(i, carry):
        base = pl.multiple_of(wid * _TPW + i * _CH, _CH)
        pltpu.sync_copy(d0_hbm.at[pl.ds(base, _CH)], i0_v)
        pltpu.sync_copy(d1_hbm.at[pl.ds(base, _CH)], i1_v)
        c0 = pltpu.async_copy(yg_hbm.at[i0_v], b0_v, sem)
        c1 = pltpu.async_copy(yg_hbm.at[i1_v], b1_v, sem)
        c0.wait()
        c1.wait()
        pltpu.sync_copy(b0_v, y0_hbm.at[pl.ds(base, _CH)])
        pltpu.sync_copy(b1_v, y1_hbm.at[pl.ds(base, _CH)])
        return carry

    jax.lax.fori_loop(0, _TPW // _CH, chunk, 0)


def _gather_sc(yg, dc0, dc1):
    f = functools.partial(
        pl.kernel,
        out_type=[
            jax.ShapeDtypeStruct((NT, HD), jnp.float32),
            jax.ShapeDtypeStruct((NT, HD), jnp.float32),
        ],
        mesh=plsc.VectorSubcoreMesh(**_SC_MESH),
        scratch_types=[
            pltpu.VMEM((_CH, HD), jnp.float32),
            pltpu.VMEM((_CH, HD), jnp.float32),
            pltpu.VMEM((_CH,), jnp.int32),
            pltpu.VMEM((_CH,), jnp.int32),
            pltpu.SemaphoreType.DMA,
        ],
    )(_gather_body)
    return f(yg, dc0, dc1)


def kernel(x, Wg, ng, Wd, nd, sWg, sng, sWd, snd, gate_norm_w, gate_w):
    xf = x.reshape(-1, HD)
    dd, dc, wt = _router(xf, gate_norm_w.reshape(1, HD), gate_w)
    xg = _dispatch(xf, dd[:, 0], dd[:, 1])
    yg = _experts(xg, Wg, ng, Wd, nd)
    y0, y1 = _gather_sc(yg, dc[:, 0], dc[:, 1])
    out = _shared_combine(xf, sWg, sng.reshape(1, HD), sWd,
                          snd.reshape(1, ID), y0, y1, wt)
    return out.reshape(x.shape[0], x.shape[1], HD)


# speedup vs baseline: 15.9236x; 1.0363x over previous
import functools

import jax
import jax.numpy as jnp
from jax.experimental import pallas as pl
from jax.experimental.pallas import tpu as pltpu
from jax.experimental.pallas import tpu_sc as plsc

HD = 1024
ID = 512
NE = 64
CAP = 80
NSLOT = NE * CAP
TRASH = NSLOT
NT = 4096
TB = 256
EPS_GATE = 1e-6
EPS_BIT = 1e-8


def _rms(x, w, eps):
    return x * jax.lax.rsqrt(jnp.mean(x * x, axis=-1, keepdims=True) + eps) * w


def _ternary_bf16(w):
    m = jnp.sum(jnp.sum(jnp.abs(w), axis=0, keepdims=True), axis=1,
                keepdims=True) * (1.0 / (w.shape[0] * w.shape[1]))
    sw = 1.0 / jnp.clip(m, 1e-5, None)
    thr = 0.5 / sw
    r = 1.0 / sw
    q = jnp.where(w > thr, r, jnp.where(w < -thr, -r, 0.0))
    return q.astype(jnp.bfloat16)


def _bit_mlp(x, wg, ng, wd, nd):
    xn = _rms(x, ng, EPS_BIT)
    sx = 127.0 / jnp.clip(jnp.max(jnp.abs(xn), axis=-1, keepdims=True), 1e-5, None)
    xq = jnp.clip(jnp.round(xn * sx), -128.0, 127.0) / sx
    y = jax.lax.dot_general(
        xq.astype(jnp.bfloat16), _ternary_bf16(wg),
        (((1,), (1,)), ((), ())), preferred_element_type=jnp.float32)
    g = y[:, :ID]
    v = y[:, ID:]
    h = g * jax.nn.sigmoid(g) * v
    hn = _rms(h, nd, EPS_BIT)
    s2 = 127.0 / jnp.clip(jnp.max(jnp.abs(hn), axis=-1, keepdims=True), 1e-5, None)
    hq = jnp.clip(jnp.round(hn * s2), -128.0, 127.0) / s2
    return jax.lax.dot_general(
        hq.astype(jnp.bfloat16), _ternary_bf16(wd),
        (((1,), (1,)), ((), ())), preferred_element_type=jnp.float32)


def _router_body(x_ref, gnw_ref, gw_ref, dd0_ref, dd1_ref, dc0_ref, dc1_ref,
                 wt_ref, cnt_ref):
    step = pl.program_id(0)

    @pl.when(step == 0)
    def _():
        cnt_ref[...] = jnp.zeros_like(cnt_ref)

    x = x_ref[...]
    xn = x * jax.lax.rsqrt(jnp.mean(x * x, axis=-1, keepdims=True) + EPS_GATE)
    xn = xn * gnw_ref[...]
    logits = jax.lax.dot_general(
        xn.astype(jnp.bfloat16), gw_ref[...].astype(jnp.bfloat16),
        (((1,), (1,)), ((), ())),
        preferred_element_type=jnp.float32)
    m = jnp.max(logits, axis=-1, keepdims=True)
    p = jnp.exp(logits - m)
    p = p / jnp.sum(p, axis=-1, keepdims=True)

    idx = jax.lax.broadcasted_iota(jnp.int32, (TB, NE), 1)
    m1 = jnp.max(p, axis=-1, keepdims=True)
    i1 = jnp.min(jnp.where(p == m1, idx, NE), axis=-1, keepdims=True)
    o1 = idx == i1
    pm = jnp.where(o1, -1.0, p)
    m2 = jnp.max(pm, axis=-1, keepdims=True)
    i2 = jnp.min(jnp.where(pm == m2, idx, NE), axis=-1, keepdims=True)
    o2 = idx == i2

    a = (o1 | o2).astype(jnp.float32)
    c = a
    sh = 1
    while sh < TB:
        c = c + jnp.concatenate(
            [jnp.zeros((sh, NE), jnp.float32), c[:-sh]], axis=0)
        sh *= 2
    carry = cnt_ref[0:1, :]
    rank = carry + c - a
    r1 = jnp.sum(jnp.where(o1, rank, 0.0), axis=-1, keepdims=True)
    r2 = jnp.sum(jnp.where(o2, rank, 0.0), axis=-1, keepdims=True)
    in1 = r1 < CAP
    in2 = r2 < CAP
    d1 = i1 * CAP + r1.astype(jnp.int32)
    d2 = i2 * CAP + r2.astype(jnp.int32)
    tok = (jax.lax.broadcasted_iota(jnp.int32, (TB, 1), 0) + step * TB)
    dd0_ref[...] = jnp.where(in1, d1, TRASH + tok)[:, 0]
    dd1_ref[...] = jnp.where(in2, d2, TRASH + tok)[:, 0]
    dc0_ref[...] = jnp.where(in1, d1, tok)[:, 0]
    dc1_ref[...] = jnp.where(in2, d2, tok)[:, 0]
    wt_ref[...] = jnp.concatenate(
        [jnp.where(in1, m1, 0.0), jnp.where(in2, m2, 0.0)], axis=1)
    cnt_ref[0:1, :] = carry + jnp.sum(a, axis=0, keepdims=True)


def _router(xf, gnw, gw):
    return pl.pallas_call(
        _router_body,
        grid=(NT // TB,),
        in_specs=[
            pl.BlockSpec((TB, HD), lambda i: (i, 0)),
            pl.BlockSpec((1, HD), lambda i: (0, 0)),
            pl.BlockSpec((NE, HD), lambda i: (0, 0)),
        ],
        out_specs=[
            pl.BlockSpec((TB,), lambda i: (i,)),
            pl.BlockSpec((TB,), lambda i: (i,)),
            pl.BlockSpec((TB,), lambda i: (i,)),
            pl.BlockSpec((TB,), lambda i: (i,)),
            pl.BlockSpec((TB, 2), lambda i: (i, 0)),
        ],
        out_shape=[
            jax.ShapeDtypeStruct((NT,), jnp.int32),
            jax.ShapeDtypeStruct((NT,), jnp.int32),
            jax.ShapeDtypeStruct((NT,), jnp.int32),
            jax.ShapeDtypeStruct((NT,), jnp.int32),
            jax.ShapeDtypeStruct((NT, 2), jnp.float32),
        ],
        scratch_shapes=[pltpu.VMEM((8, NE), jnp.float32)],
    )(xf, gnw, gw)


def _expert_body(xg_ref, wg_ref, ng_ref, wd_ref, nd_ref, yg_ref):
    yg_ref[...] = _bit_mlp(
        xg_ref[...], wg_ref[0], ng_ref[0], wd_ref[0], nd_ref[0])


def _experts(xg, Wg, ng, Wd, nd):
    return pl.pallas_call(
        _expert_body,
        grid=(NE,),
        in_specs=[
            pl.BlockSpec((CAP, HD), lambda e: (e, 0)),
            pl.BlockSpec((1, 2 * ID, HD), lambda e: (e, 0, 0)),
            pl.BlockSpec((1, 1, HD), lambda e: (e, 0, 0)),
            pl.BlockSpec((1, HD, ID), lambda e: (e, 0, 0)),
            pl.BlockSpec((1, 1, ID), lambda e: (e, 0, 0)),
        ],
        out_specs=pl.BlockSpec((CAP, HD), lambda e: (e, 0)),
        out_shape=jax.ShapeDtypeStruct((NSLOT, HD), jnp.float32),
    )(xg, Wg, ng.reshape(NE, 1, HD), Wd, nd.reshape(NE, 1, ID))


def _quant_body(wg_ref, wd_ref, wgq_ref, wdq_ref):
    wgq_ref[...] = _ternary_bf16(wg_ref[...])
    wdq_ref[...] = _ternary_bf16(wd_ref[...])


def _quant_shared(sWg, sWd):
    return pl.pallas_call(
        _quant_body,
        out_shape=[
            jax.ShapeDtypeStruct((2 * ID, HD), jnp.bfloat16),
            jax.ShapeDtypeStruct((HD, ID), jnp.bfloat16),
        ],
    )(sWg, sWd)


def _shared_body(x_ref, wgq_ref, ng_ref, wdq_ref, nd_ref,
                 y0_ref, y1_ref, wt_ref, y_ref):
    x = x_ref[...]
    ng = ng_ref[...]
    nd = nd_ref[...]
    xn = _rms(x, ng, EPS_BIT)
    sx = 127.0 / jnp.clip(jnp.max(jnp.abs(xn), axis=-1, keepdims=True), 1e-5, None)
    xq = jnp.clip(jnp.round(xn * sx), -128.0, 127.0) / sx
    y = jax.lax.dot_general(
        xq.astype(jnp.bfloat16), wgq_ref[...],
        (((1,), (1,)), ((), ())), preferred_element_type=jnp.float32)
    g = y[:, :ID]
    v = y[:, ID:]
    h = g * jax.nn.sigmoid(g) * v
    hn = _rms(h, nd, EPS_BIT)
    s2 = 127.0 / jnp.clip(jnp.max(jnp.abs(hn), axis=-1, keepdims=True), 1e-5, None)
    hq = jnp.clip(jnp.round(hn * s2), -128.0, 127.0) / s2
    sh = jax.lax.dot_general(
        hq.astype(jnp.bfloat16), wdq_ref[...],
        (((1,), (1,)), ((), ())), preferred_element_type=jnp.float32)
    w0 = wt_ref[:, 0:1]
    w1 = wt_ref[:, 1:2]
    z = jnp.zeros_like(sh)
    o = sh + jnp.where(w0 != 0.0, w0 * y0_ref[...], z)
    y_ref[...] = o + jnp.where(w1 != 0.0, w1 * y1_ref[...], z)


def _shared_combine(xf, sWg, sng, sWd, snd, y0, y1, wt):
    blk = 512
    wgq, wdq = _quant_shared(sWg, sWd)
    return pl.pallas_call(
        _shared_body,
        grid=(NT // blk,),
        in_specs=[
            pl.BlockSpec((blk, HD), lambda i: (i, 0)),
            pl.BlockSpec((2 * ID, HD), lambda i: (0, 0)),
            pl.BlockSpec((1, HD), lambda i: (0, 0)),
            pl.BlockSpec((HD, ID), lambda i: (0, 0)),
            pl.BlockSpec((1, ID), lambda i: (0, 0)),
            pl.BlockSpec((blk, HD), lambda i: (i, 0)),
            pl.BlockSpec((blk, HD), lambda i: (i, 0)),
            pl.BlockSpec((blk, 2), lambda i: (i, 0)),
        ],
        out_specs=pl.BlockSpec((blk, HD), lambda i: (i, 0)),
        out_shape=jax.ShapeDtypeStruct((NT, HD), jnp.float32),
    )(xf, wgq, sng, wdq, snd, y0, y1, wt)


_NW = 32
_TPW = NT // _NW
_CH = 32

_SC_MESH = dict(core_axis_name="c", subcore_axis_name="s",
                num_cores=2, num_subcores=16)


def _worker_id():
    return jax.lax.axis_index("s") * 2 + jax.lax.axis_index("c")


def _dispatch_body(x_hbm, d0_hbm, d1_hbm, xg_hbm,
                   rows_v, i0a, i1a, i0b, i1b, sem):
    wid = _worker_id()
    base = pl.multiple_of(wid * _TPW, 64)
    pltpu.sync_copy(d0_hbm.at[pl.ds(base, 64)], i0a)
    pltpu.sync_copy(d1_hbm.at[pl.ds(base, 64)], i1a)
    pltpu.sync_copy(d0_hbm.at[pl.ds(base + 64, 64)], i0b)
    pltpu.sync_copy(d1_hbm.at[pl.ds(base + 64, 64)], i1b)
    for c, i0, i1 in ((0, i0a, i1a), (1, i0b, i1b)):
        b = pl.multiple_of(base + c * 64, 64)
        pltpu.sync_copy(x_hbm.at[pl.ds(b, 64)], rows_v)
        c0 = pltpu.async_copy(rows_v, xg_hbm.at[i0], sem)
        c1 = pltpu.async_copy(rows_v, xg_hbm.at[i1], sem)
        c0.wait()
        c1.wait()


def _dispatch(xf, dd0, dd1):
    f = functools.partial(
        pl.kernel,
        out_type=jax.ShapeDtypeStruct((NSLOT + NT, HD), jnp.float32),
        mesh=plsc.VectorSubcoreMesh(**_SC_MESH),
        scratch_types=[
            pltpu.VMEM((64, HD), jnp.float32),
            pltpu.VMEM((64,), jnp.int32),
            pltpu.VMEM((64,), jnp.int32),
            pltpu.VMEM((64,), jnp.int32),
            pltpu.VMEM((64,), jnp.int32),
            pltpu.SemaphoreType.DMA,
        ],
    )(_dispatch_body)
    return f(xf, dd0, dd1)


def _gather_body(yg_hbm, d0_hbm, d1_hbm, y0_hbm, y1_hbm,
                 b0_v, b1_v, i0_v, i1_v, sem):
    wid = _worker_id()

    def chunk(i, carry):
        base = pl.multiple_of(wid * _TPW + i * _CH, _CH)
        pltpu.sync_copy(d0_hbm.at[pl.ds(base, _CH)], i0_v)
        pltpu.sync_copy(d1_hbm.at[pl.ds(base, _CH)], i1_v)
        c0 = pltpu.async_copy(yg_hbm.at[i0_v], b0_v, sem)
        c1 = pltpu.async_copy(yg_hbm.at[i1_v], b1_v, sem)
        c0.wait()
        c1.wait()
        pltpu.sync_copy(b0_v, y0_hbm.at[pl.ds(base, _CH)])
        pltpu.sync_copy(b1_v, y1_hbm.at[pl.ds(base, _CH)])
        return carry

    jax.lax.fori_loop(0, _TPW // _CH, chunk, 0)


def _gather_sc(yg, dc0, dc1):
    f = functools.partial(
        pl.kernel,
        out_type=[
            jax.ShapeDtypeStruct((NT, HD), jnp.float32),
            jax.ShapeDtypeStruct((NT, HD), jnp.float32),
        ],
        mesh=plsc.VectorSubcoreMesh(**_SC_MESH),
        scratch_types=[
            pltpu.VMEM((_CH, HD), jnp.float32),
            pltpu.VMEM((_CH, HD), jnp.float32),
            pltpu.VMEM((_CH,), jnp.int32),
            pltpu.VMEM((_CH,), jnp.int32),
            pltpu.SemaphoreType.DMA,
        ],
    )(_gather_body)
    return f(yg, dc0, dc1)


def kernel(x, Wg, ng, Wd, nd, sWg, sng, sWd, snd, gate_norm_w, gate_w):
    xf = x.reshape(-1, HD)
    dd0, dd1, dc0, dc1, wt = _router(xf, gate_norm_w.reshape(1, HD), gate_w)
    xg = _dispatch(xf, dd0, dd1)
    yg = _experts(xg, Wg, ng, Wd, nd)
    y0, y1 = _gather_sc(yg, dc0, dc1)
    out = _shared_combine(xf, sWg, sng.reshape(1, HD), sWd,
                          snd.reshape(1, ID), y0, y1, wt)
    return out.reshape(x.shape[0], x.shape[1], HD)


# speedup vs baseline: 15.9677x; 1.0028x over previous
import functools

import jax
import jax.numpy as jnp
from jax.experimental import pallas as pl
from jax.experimental.pallas import tpu as pltpu
from jax.experimental.pallas import tpu_sc as plsc

HD = 1024
ID = 512
NE = 64
CAP = 80
NSLOT = NE * CAP
TRASH = NSLOT
NT = 4096
TB = 256
EPS_GATE = 1e-6
EPS_BIT = 1e-8


def _rms(x, w, eps):
    return x * jax.lax.rsqrt(jnp.mean(x * x, axis=-1, keepdims=True) + eps) * w


def _ternary_bf16(w):
    m = jnp.sum(jnp.sum(jnp.abs(w), axis=0, keepdims=True), axis=1,
                keepdims=True) * (1.0 / (w.shape[0] * w.shape[1]))
    sw = 1.0 / jnp.clip(m, 1e-5, None)
    thr = 0.5 / sw
    r = 1.0 / sw
    q = jnp.where(w > thr, r, jnp.where(w < -thr, -r, 0.0))
    return q.astype(jnp.bfloat16)


def _bit_mlp(x, wg, ng, wd, nd):
    xn = _rms(x, ng, EPS_BIT)
    sx = 127.0 / jnp.clip(jnp.max(jnp.abs(xn), axis=-1, keepdims=True), 1e-5, None)
    xq = jnp.clip(jnp.round(xn * sx), -128.0, 127.0) / sx
    y = jax.lax.dot_general(
        xq.astype(jnp.bfloat16), _ternary_bf16(wg),
        (((1,), (1,)), ((), ())), preferred_element_type=jnp.float32)
    g = y[:, :ID]
    v = y[:, ID:]
    h = g * jax.nn.sigmoid(g) * v
    hn = _rms(h, nd, EPS_BIT)
    s2 = 127.0 / jnp.clip(jnp.max(jnp.abs(hn), axis=-1, keepdims=True), 1e-5, None)
    hq = jnp.clip(jnp.round(hn * s2), -128.0, 127.0) / s2
    return jax.lax.dot_general(
        hq.astype(jnp.bfloat16), _ternary_bf16(wd),
        (((1,), (1,)), ((), ())), preferred_element_type=jnp.float32)


def _router_body(x_ref, gnw_ref, gw_ref, dd0_ref, dd1_ref, dc0_ref, dc1_ref,
                 wt_ref, cnt_ref):
    step = pl.program_id(0)

    @pl.when(step == 0)
    def _():
        cnt_ref[...] = jnp.zeros_like(cnt_ref)

    x = x_ref[...]
    xn = x * jax.lax.rsqrt(jnp.mean(x * x, axis=-1, keepdims=True) + EPS_GATE)
    xn = xn * gnw_ref[...]
    logits = jax.lax.dot_general(
        xn.astype(jnp.bfloat16), gw_ref[...].astype(jnp.bfloat16),
        (((1,), (1,)), ((), ())),
        preferred_element_type=jnp.float32)
    m = jnp.max(logits, axis=-1, keepdims=True)
    p = jnp.exp(logits - m)
    p = p / jnp.sum(p, axis=-1, keepdims=True)

    idx = jax.lax.broadcasted_iota(jnp.int32, (TB, NE), 1)
    m1 = jnp.max(p, axis=-1, keepdims=True)
    i1 = jnp.min(jnp.where(p == m1, idx, NE), axis=-1, keepdims=True)
    o1 = idx == i1
    pm = jnp.where(o1, -1.0, p)
    m2 = jnp.max(pm, axis=-1, keepdims=True)
    i2 = jnp.min(jnp.where(pm == m2, idx, NE), axis=-1, keepdims=True)
    o2 = idx == i2

    a = (o1 | o2).astype(jnp.float32)
    c = a
    sh = 1
    while sh < TB:
        c = c + jnp.concatenate(
            [jnp.zeros((sh, NE), jnp.float32), c[:-sh]], axis=0)
        sh *= 2
    carry = cnt_ref[0:1, :]
    rank = carry + c - a
    r1 = jnp.sum(jnp.where(o1, rank, 0.0), axis=-1, keepdims=True)
    r2 = jnp.sum(jnp.where(o2, rank, 0.0), axis=-1, keepdims=True)
    in1 = r1 < CAP
    in2 = r2 < CAP
    d1 = i1 * CAP + r1.astype(jnp.int32)
    d2 = i2 * CAP + r2.astype(jnp.int32)
    tok = (jax.lax.broadcasted_iota(jnp.int32, (TB, 1), 0) + step * TB)
    dd0_ref[...] = jnp.where(in1, d1, TRASH + tok)[:, 0]
    dd1_ref[...] = jnp.where(in2, d2, TRASH + tok)[:, 0]
    dc0_ref[...] = jnp.where(in1, d1, tok)[:, 0]
    dc1_ref[...] = jnp.where(in2, d2, tok)[:, 0]
    wt_ref[...] = jnp.concatenate(
        [jnp.where(in1, m1, 0.0), jnp.where(in2, m2, 0.0)], axis=1)
    cnt_ref[0:1, :] = carry + jnp.sum(a, axis=0, keepdims=True)


def _router(xf, gnw, gw):
    return pl.pallas_call(
        _router_body,
        grid=(NT // TB,),
        in_specs=[
            pl.BlockSpec((TB, HD), lambda i: (i, 0)),
            pl.BlockSpec((1, HD), lambda i: (0, 0)),
            pl.BlockSpec((NE, HD), lambda i: (0, 0)),
        ],
        out_specs=[
            pl.BlockSpec((TB,), lambda i: (i,)),
            pl.BlockSpec((TB,), lambda i: (i,)),
            pl.BlockSpec((TB,), lambda i: (i,)),
            pl.BlockSpec((TB,), lambda i: (i,)),
            pl.BlockSpec((TB, 2), lambda i: (i, 0)),
        ],
        out_shape=[
            jax.ShapeDtypeStruct((NT,), jnp.int32),
            jax.ShapeDtypeStruct((NT,), jnp.int32),
            jax.ShapeDtypeStruct((NT,), jnp.int32),
            jax.ShapeDtypeStruct((NT,), jnp.int32),
            jax.ShapeDtypeStruct((NT, 2), jnp.float32),
        ],
        scratch_shapes=[pltpu.VMEM((8, NE), jnp.float32)],
    )(xf, gnw, gw)


def _expert_body(xg_ref, wg_ref, ng_ref, wd_ref, nd_ref, yg_ref):
    yg_ref[...] = _bit_mlp(
        xg_ref[...], wg_ref[0], ng_ref[0], wd_ref[0], nd_ref[0])


def _experts(xg, Wg, ng, Wd, nd):
    return pl.pallas_call(
        _expert_body,
        grid=(NE,),
        in_specs=[
            pl.BlockSpec((CAP, HD), lambda e: (e, 0)),
            pl.BlockSpec((1, 2 * ID, HD), lambda e: (e, 0, 0)),
            pl.BlockSpec((1, 1, HD), lambda e: (e, 0, 0)),
            pl.BlockSpec((1, HD, ID), lambda e: (e, 0, 0)),
            pl.BlockSpec((1, 1, ID), lambda e: (e, 0, 0)),
        ],
        out_specs=pl.BlockSpec((CAP, HD), lambda e: (e, 0)),
        out_shape=jax.ShapeDtypeStruct((NSLOT, HD), jnp.float32),
    )(xg, Wg, ng.reshape(NE, 1, HD), Wd, nd.reshape(NE, 1, ID))


def _shared_body(x_ref, wg_ref, ng_ref, wd_ref, nd_ref,
                 y0_ref, y1_ref, wt_ref, y_ref, wgq_ref, wdq_ref):
    @pl.when(pl.program_id(0) == 0)
    def _():
        wgq_ref[...] = _ternary_bf16(wg_ref[...])
        wdq_ref[...] = _ternary_bf16(wd_ref[...])

    x = x_ref[...]
    ng = ng_ref[...]
    nd = nd_ref[...]
    xn = _rms(x, ng, EPS_BIT)
    sx = 127.0 / jnp.clip(jnp.max(jnp.abs(xn), axis=-1, keepdims=True), 1e-5, None)
    xq = jnp.clip(jnp.round(xn * sx), -128.0, 127.0) / sx
    y = jax.lax.dot_general(
        xq.astype(jnp.bfloat16), wgq_ref[...],
        (((1,), (1,)), ((), ())), preferred_element_type=jnp.float32)
    g = y[:, :ID]
    v = y[:, ID:]
    h = g * jax.nn.sigmoid(g) * v
    hn = _rms(h, nd, EPS_BIT)
    s2 = 127.0 / jnp.clip(jnp.max(jnp.abs(hn), axis=-1, keepdims=True), 1e-5, None)
    hq = jnp.clip(jnp.round(hn * s2), -128.0, 127.0) / s2
    sh = jax.lax.dot_general(
        hq.astype(jnp.bfloat16), wdq_ref[...],
        (((1,), (1,)), ((), ())), preferred_element_type=jnp.float32)
    w0 = wt_ref[:, 0:1]
    w1 = wt_ref[:, 1:2]
    z = jnp.zeros_like(sh)
    o = sh + jnp.where(w0 != 0.0, w0 * y0_ref[...], z)
    y_ref[...] = o + jnp.where(w1 != 0.0, w1 * y1_ref[...], z)


def _shared_combine(xf, sWg, sng, sWd, snd, y0, y1, wt):
    blk = 512
    return pl.pallas_call(
        _shared_body,
        grid=(NT // blk,),
        in_specs=[
            pl.BlockSpec((blk, HD), lambda i: (i, 0)),
            pl.BlockSpec((2 * ID, HD), lambda i: (0, 0)),
            pl.BlockSpec((1, HD), lambda i: (0, 0)),
            pl.BlockSpec((HD, ID), lambda i: (0, 0)),
            pl.BlockSpec((1, ID), lambda i: (0, 0)),
            pl.BlockSpec((blk, HD), lambda i: (i, 0)),
            pl.BlockSpec((blk, HD), lambda i: (i, 0)),
            pl.BlockSpec((blk, 2), lambda i: (i, 0)),
        ],
        out_specs=pl.BlockSpec((blk, HD), lambda i: (i, 0)),
        out_shape=jax.ShapeDtypeStruct((NT, HD), jnp.float32),
        scratch_shapes=[
            pltpu.VMEM((2 * ID, HD), jnp.bfloat16),
            pltpu.VMEM((HD, ID), jnp.bfloat16),
        ],
    )(xf, sWg, sng, sWd, snd, y0, y1, wt)


_NW = 32
_TPW = NT // _NW
_CH = 32

_SC_MESH = dict(core_axis_name="c", subcore_axis_name="s",
                num_cores=2, num_subcores=16)


def _worker_id():
    return jax.lax.axis_index("s") * 2 + jax.lax.axis_index("c")


def _dispatch_body(x_hbm, d0_hbm, d1_hbm, xg_hbm,
                   rows_v, i0a, i1a, i0b, i1b, sem):
    wid = _worker_id()
    base = pl.multiple_of(wid * _TPW, 64)
    pltpu.sync_copy(d0_hbm.at[pl.ds(base, 64)], i0a)
    pltpu.sync_copy(d1_hbm.at[pl.ds(base, 64)], i1a)
    pltpu.sync_copy(d0_hbm.at[pl.ds(base + 64, 64)], i0b)
    pltpu.sync_copy(d1_hbm.at[pl.ds(base + 64, 64)], i1b)
    for c, i0, i1 in ((0, i0a, i1a), (1, i0b, i1b)):
        b = pl.multiple_of(base + c * 64, 64)
        pltpu.sync_copy(x_hbm.at[pl.ds(b, 64)], rows_v)
        c0 = pltpu.async_copy(rows_v, xg_hbm.at[i0], sem)
        c1 = pltpu.async_copy(rows_v, xg_hbm.at[i1], sem)
        c0.wait()
        c1.wait()


def _dispatch(xf, dd0, dd1):
    f = functools.partial(
        pl.kernel,
        out_type=jax.ShapeDtypeStruct((NSLOT + NT, HD), jnp.float32),
        mesh=plsc.VectorSubcoreMesh(**_SC_MESH),
        scratch_types=[
            pltpu.VMEM((64, HD), jnp.float32),
            pltpu.VMEM((64,), jnp.int32),
            pltpu.VMEM((64,), jnp.int32),
            pltpu.VMEM((64,), jnp.int32),
            pltpu.VMEM((64,), jnp.int32),
            pltpu.SemaphoreType.DMA,
        ],
    )(_dispatch_body)
    return f(xf, dd0, dd1)


def _gather_body(yg_hbm, d0_hbm, d1_hbm, y0_hbm, y1_hbm,
                 b0_v, b1_v, i0_v, i1_v, sem):
    wid = _worker_id()

    def chunk(i, carry):
        base = pl.multiple_of(wid * _TPW + i * _CH, _CH)
        pltpu.sync_copy(d0_hbm.at[pl.ds(base, _CH)], i0_v)
        pltpu.sync_copy(d1_hbm.at[pl.ds(base, _CH)], i1_v)
        c0 = pltpu.async_copy(yg_hbm.at[i0_v], b0_v, sem)
        c1 = pltpu.async_copy(yg_hbm.at[i1_v], b1_v, sem)
        c0.wait()
        c1.wait()
        pltpu.sync_copy(b0_v, y0_hbm.at[pl.ds(base, _CH)])
        pltpu.sync_copy(b1_v, y1_hbm.at[pl.ds(base, _CH)])
        return carry

    jax.lax.fori_loop(0, _TPW // _CH, chunk, 0)


def _gather_sc(yg, dc0, dc1):
    f = functools.partial(
        pl.kernel,
        out_type=[
            jax.ShapeDtypeStruct((NT, HD), jnp.float32),
            jax.ShapeDtypeStruct((NT, HD), jnp.float32),
        ],
        mesh=plsc.VectorSubcoreMesh(**_SC_MESH),
        scratch_types=[
            pltpu.VMEM((_CH, HD), jnp.float32),
            pltpu.VMEM((_CH, HD), jnp.float32),
            pltpu.VMEM((_CH,), jnp.int32),
            pltpu.VMEM((_CH,), jnp.int32),
            pltpu.SemaphoreType.DMA,
        ],
    )(_gather_body)
    return f(yg, dc0, dc1)


def kernel(x, Wg, ng, Wd, nd, sWg, sng, sWd, snd, gate_norm_w, gate_w):
    xf = x.reshape(-1, HD)
    dd0, dd1, dc0, dc1, wt = _router(xf, gate_norm_w.reshape(1, HD), gate_w)
    xg = _dispatch(xf, dd0, dd1)
    yg = _experts(xg, Wg, ng, Wd, nd)
    y0, y1 = _gather_sc(yg, dc0, dc1)
    out = _shared_combine(xf, sWg, sng.reshape(1, HD), sWd,
                          snd.reshape(1, ID), y0, y1, wt)
    return out.reshape(x.shape[0], x.shape[1], HD)


# speedup vs baseline: 16.2818x; 1.0197x over previous
import functools

import jax
import jax.numpy as jnp
from jax.experimental import pallas as pl
from jax.experimental.pallas import tpu as pltpu
from jax.experimental.pallas import tpu_sc as plsc

HD = 1024
ID = 512
NE = 64
CAP = 80
NSLOT = NE * CAP
TRASH = NSLOT
NT = 4096
TB = 512
EPS_GATE = 1e-6
EPS_BIT = 1e-8


def _rms(x, w, eps):
    return x * jax.lax.rsqrt(jnp.mean(x * x, axis=-1, keepdims=True) + eps) * w


def _ternary_bf16(w):
    m = jnp.sum(jnp.sum(jnp.abs(w), axis=0, keepdims=True), axis=1,
                keepdims=True) * (1.0 / (w.shape[0] * w.shape[1]))
    sw = 1.0 / jnp.clip(m, 1e-5, None)
    thr = 0.5 / sw
    r = 1.0 / sw
    q = jnp.where(w > thr, r, jnp.where(w < -thr, -r, 0.0))
    return q.astype(jnp.bfloat16)


def _bit_mlp(x, wg, ng, wd, nd):
    xn = _rms(x, ng, EPS_BIT)
    sx = 127.0 / jnp.clip(jnp.max(jnp.abs(xn), axis=-1, keepdims=True), 1e-5, None)
    xq = jnp.clip(jnp.round(xn * sx), -128.0, 127.0) / sx
    y = jax.lax.dot_general(
        xq.astype(jnp.bfloat16), _ternary_bf16(wg),
        (((1,), (1,)), ((), ())), preferred_element_type=jnp.float32)
    g = y[:, :ID]
    v = y[:, ID:]
    h = g * jax.nn.sigmoid(g) * v
    hn = _rms(h, nd, EPS_BIT)
    s2 = 127.0 / jnp.clip(jnp.max(jnp.abs(hn), axis=-1, keepdims=True), 1e-5, None)
    hq = jnp.clip(jnp.round(hn * s2), -128.0, 127.0) / s2
    return jax.lax.dot_general(
        hq.astype(jnp.bfloat16), _ternary_bf16(wd),
        (((1,), (1,)), ((), ())), preferred_element_type=jnp.float32)


def _router_body(x_ref, gnw_ref, gw_ref, dd0_ref, dd1_ref, dc0_ref, dc1_ref,
                 wt_ref, cnt_ref):
    step = pl.program_id(0)

    @pl.when(step == 0)
    def _():
        cnt_ref[...] = jnp.zeros_like(cnt_ref)

    x = x_ref[...]
    xn = x * jax.lax.rsqrt(jnp.mean(x * x, axis=-1, keepdims=True) + EPS_GATE)
    xn = xn * gnw_ref[...]
    logits = jax.lax.dot_general(
        xn.astype(jnp.bfloat16), gw_ref[...].astype(jnp.bfloat16),
        (((1,), (1,)), ((), ())),
        preferred_element_type=jnp.float32)
    m = jnp.max(logits, axis=-1, keepdims=True)
    p = jnp.exp(logits - m)
    p = p / jnp.sum(p, axis=-1, keepdims=True)

    idx = jax.lax.broadcasted_iota(jnp.int32, (TB, NE), 1)
    m1 = jnp.max(p, axis=-1, keepdims=True)
    i1 = jnp.min(jnp.where(p == m1, idx, NE), axis=-1, keepdims=True)
    o1 = idx == i1
    pm = jnp.where(o1, -1.0, p)
    m2 = jnp.max(pm, axis=-1, keepdims=True)
    i2 = jnp.min(jnp.where(pm == m2, idx, NE), axis=-1, keepdims=True)
    o2 = idx == i2

    a = (o1 | o2).astype(jnp.float32)
    c = a
    sh = 1
    while sh < TB:
        c = c + jnp.concatenate(
            [jnp.zeros((sh, NE), jnp.float32), c[:-sh]], axis=0)
        sh *= 2
    carry = cnt_ref[0:1, :]
    rank = carry + c - a
    r1 = jnp.sum(jnp.where(o1, rank, 0.0), axis=-1, keepdims=True)
    r2 = jnp.sum(jnp.where(o2, rank, 0.0), axis=-1, keepdims=True)
    in1 = r1 < CAP
    in2 = r2 < CAP
    d1 = i1 * CAP + r1.astype(jnp.int32)
    d2 = i2 * CAP + r2.astype(jnp.int32)
    tok = (jax.lax.broadcasted_iota(jnp.int32, (TB, 1), 0) + step * TB)
    dd0_ref[...] = jnp.where(in1, d1, TRASH + tok)[:, 0]
    dd1_ref[...] = jnp.where(in2, d2, TRASH + tok)[:, 0]
    dc0_ref[...] = jnp.where(in1, d1, tok)[:, 0]
    dc1_ref[...] = jnp.where(in2, d2, tok)[:, 0]
    wt_ref[...] = jnp.concatenate(
        [jnp.where(in1, m1, 0.0), jnp.where(in2, m2, 0.0)], axis=1)
    cnt_ref[0:1, :] = carry + jnp.sum(a, axis=0, keepdims=True)


def _router(xf, gnw, gw):
    return pl.pallas_call(
        _router_body,
        grid=(NT // TB,),
        in_specs=[
            pl.BlockSpec((TB, HD), lambda i: (i, 0)),
            pl.BlockSpec((1, HD), lambda i: (0, 0)),
            pl.BlockSpec((NE, HD), lambda i: (0, 0)),
        ],
        out_specs=[
            pl.BlockSpec((TB,), lambda i: (i,)),
            pl.BlockSpec((TB,), lambda i: (i,)),
            pl.BlockSpec((TB,), lambda i: (i,)),
            pl.BlockSpec((TB,), lambda i: (i,)),
            pl.BlockSpec((TB, 2), lambda i: (i, 0)),
        ],
        out_shape=[
            jax.ShapeDtypeStruct((NT,), jnp.int32),
            jax.ShapeDtypeStruct((NT,), jnp.int32),
            jax.ShapeDtypeStruct((NT,), jnp.int32),
            jax.ShapeDtypeStruct((NT,), jnp.int32),
            jax.ShapeDtypeStruct((NT, 2), jnp.float32),
        ],
        scratch_shapes=[pltpu.VMEM((8, NE), jnp.float32)],
    )(xf, gnw, gw)


def _expert_body(xg_ref, wg_ref, ng_ref, wd_ref, nd_ref, yg_ref):
    yg_ref[...] = _bit_mlp(
        xg_ref[...], wg_ref[0], ng_ref[0], wd_ref[0], nd_ref[0])


def _experts(xg, Wg, ng, Wd, nd):
    return pl.pallas_call(
        _expert_body,
        grid=(NE,),
        in_specs=[
            pl.BlockSpec((CAP, HD), lambda e: (e, 0)),
            pl.BlockSpec((1, 2 * ID, HD), lambda e: (e, 0, 0)),
            pl.BlockSpec((1, 1, HD), lambda e: (e, 0, 0)),
            pl.BlockSpec((1, HD, ID), lambda e: (e, 0, 0)),
            pl.BlockSpec((1, 1, ID), lambda e: (e, 0, 0)),
        ],
        out_specs=pl.BlockSpec((CAP, HD), lambda e: (e, 0)),
        out_shape=jax.ShapeDtypeStruct((NSLOT, HD), jnp.float32),
    )(xg, Wg, ng.reshape(NE, 1, HD), Wd, nd.reshape(NE, 1, ID))


def _shared_body(x_ref, wg_ref, ng_ref, wd_ref, nd_ref,
                 y0_ref, y1_ref, wt_ref, y_ref, wgq_ref, wdq_ref):
    @pl.when(pl.program_id(0) == 0)
    def _():
        wgq_ref[...] = _ternary_bf16(wg_ref[...])
        wdq_ref[...] = _ternary_bf16(wd_ref[...])

    x = x_ref[...]
    ng = ng_ref[...]
    nd = nd_ref[...]
    xn = _rms(x, ng, EPS_BIT)
    sx = 127.0 / jnp.clip(jnp.max(jnp.abs(xn), axis=-1, keepdims=True), 1e-5, None)
    xq = jnp.clip(jnp.round(xn * sx), -128.0, 127.0) / sx
    y = jax.lax.dot_general(
        xq.astype(jnp.bfloat16), wgq_ref[...],
        (((1,), (1,)), ((), ())), preferred_element_type=jnp.float32)
    g = y[:, :ID]
    v = y[:, ID:]
    h = g * jax.nn.sigmoid(g) * v
    hn = _rms(h, nd, EPS_BIT)
    s2 = 127.0 / jnp.clip(jnp.max(jnp.abs(hn), axis=-1, keepdims=True), 1e-5, None)
    hq = jnp.clip(jnp.round(hn * s2), -128.0, 127.0) / s2
    sh = jax.lax.dot_general(
        hq.astype(jnp.bfloat16), wdq_ref[...],
        (((1,), (1,)), ((), ())), preferred_element_type=jnp.float32)
    w0 = wt_ref[:, 0:1]
    w1 = wt_ref[:, 1:2]
    z = jnp.zeros_like(sh)
    o = sh + jnp.where(w0 != 0.0, w0 * y0_ref[...], z)
    y_ref[...] = o + jnp.where(w1 != 0.0, w1 * y1_ref[...], z)


def _shared_combine(xf, sWg, sng, sWd, snd, y0, y1, wt):
    blk = 512
    return pl.pallas_call(
        _shared_body,
        grid=(NT // blk,),
        in_specs=[
            pl.BlockSpec((blk, HD), lambda i: (i, 0)),
            pl.BlockSpec((2 * ID, HD), lambda i: (0, 0)),
            pl.BlockSpec((1, HD), lambda i: (0, 0)),
            pl.BlockSpec((HD, ID), lambda i: (0, 0)),
            pl.BlockSpec((1, ID), lambda i: (0, 0)),
            pl.BlockSpec((blk, HD), lambda i: (i, 0)),
            pl.BlockSpec((blk, HD), lambda i: (i, 0)),
            pl.BlockSpec((blk, 2), lambda i: (i, 0)),
        ],
        out_specs=pl.BlockSpec((blk, HD), lambda i: (i, 0)),
        out_shape=jax.ShapeDtypeStruct((NT, HD), jnp.float32),
        scratch_shapes=[
            pltpu.VMEM((2 * ID, HD), jnp.bfloat16),
            pltpu.VMEM((HD, ID), jnp.bfloat16),
        ],
    )(xf, sWg, sng, sWd, snd, y0, y1, wt)


_NW = 32
_TPW = NT // _NW
_CH = 32

_SC_MESH = dict(core_axis_name="c", subcore_axis_name="s",
                num_cores=2, num_subcores=16)


def _worker_id():
    return jax.lax.axis_index("s") * 2 + jax.lax.axis_index("c")


def _dispatch_body(x_hbm, d0_hbm, d1_hbm, xg_hbm,
                   rows_v, i0a, i1a, i0b, i1b, sem):
    wid = _worker_id()
    base = pl.multiple_of(wid * _TPW, 64)
    pltpu.sync_copy(d0_hbm.at[pl.ds(base, 64)], i0a)
    pltpu.sync_copy(d1_hbm.at[pl.ds(base, 64)], i1a)
    pltpu.sync_copy(d0_hbm.at[pl.ds(base + 64, 64)], i0b)
    pltpu.sync_copy(d1_hbm.at[pl.ds(base + 64, 64)], i1b)
    for c, i0, i1 in ((0, i0a, i1a), (1, i0b, i1b)):
        b = pl.multiple_of(base + c * 64, 64)
        pltpu.sync_copy(x_hbm.at[pl.ds(b, 64)], rows_v)
        c0 = pltpu.async_copy(rows_v, xg_hbm.at[i0], sem)
        c1 = pltpu.async_copy(rows_v, xg_hbm.at[i1], sem)
        c0.wait()
        c1.wait()


def _dispatch(xf, dd0, dd1):
    f = functools.partial(
        pl.kernel,
        out_type=jax.ShapeDtypeStruct((NSLOT + NT, HD), jnp.float32),
        mesh=plsc.VectorSubcoreMesh(**_SC_MESH),
        scratch_types=[
            pltpu.VMEM((64, HD), jnp.float32),
            pltpu.VMEM((64,), jnp.int32),
            pltpu.VMEM((64,), jnp.int32),
            pltpu.VMEM((64,), jnp.int32),
            pltpu.VMEM((64,), jnp.int32),
            pltpu.SemaphoreType.DMA,
        ],
    )(_dispatch_body)
    return f(xf, dd0, dd1)


def _gather_body(yg_hbm, d0_hbm, d1_hbm, y0_hbm, y1_hbm,
                 b0_v, b1_v, i0_v, i1_v, sem):
    wid = _worker_id()

    def chunk(i, carry):
        base = pl.multiple_of(wid * _TPW + i * _CH, _CH)
        pltpu.sync_copy(d0_hbm.at[pl.ds(base, _CH)], i0_v)
        pltpu.sync_copy(d1_hbm.at[pl.ds(base, _CH)], i1_v)
        c0 = pltpu.async_copy(yg_hbm.at[i0_v], b0_v, sem)
        c1 = pltpu.async_copy(yg_hbm.at[i1_v], b1_v, sem)
        c0.wait()
        c1.wait()
        pltpu.sync_copy(b0_v, y0_hbm.at[pl.ds(base, _CH)])
        pltpu.sync_copy(b1_v, y1_hbm.at[pl.ds(base, _CH)])
        return carry

    jax.lax.fori_loop(0, _TPW // _CH, chunk, 0)


def _gather_sc(yg, dc0, dc1):
    f = functools.partial(
        pl.kernel,
        out_type=[
            jax.ShapeDtypeStruct((NT, HD), jnp.float32),
            jax.ShapeDtypeStruct((NT, HD), jnp.float32),
        ],
        mesh=plsc.VectorSubcoreMesh(**_SC_MESH),
        scratch_types=[
            pltpu.VMEM((_CH, HD), jnp.float32),
            pltpu.VMEM((_CH, HD), jnp.float32),
            pltpu.VMEM((_CH,), jnp.int32),
            pltpu.VMEM((_CH,), jnp.int32),
            pltpu.SemaphoreType.DMA,
        ],
    )(_gather_body)
    return f(yg, dc0, dc1)


def kernel(x, Wg, ng, Wd, nd, sWg, sng, sWd, snd, gate_norm_w, gate_w):
    xf = x.reshape(-1, HD)
    dd0, dd1, dc0, dc1, wt = _router(xf, gate_norm_w.reshape(1, HD), gate_w)
    xg = _dispatch(xf, dd0, dd1)
    yg = _experts(xg, Wg, ng, Wd, nd)
    y0, y1 = _gather_sc(yg, dc0, dc1)
    out = _shared_combine(xf, sWg, sng.reshape(1, HD), sWd,
                          snd.reshape(1, ID), y0, y1, wt)
    return out.reshape(x.shape[0], x.shape[1], HD)
